# baseline (speedup 1.0000x reference)
import functools

import jax
import jax.numpy as jnp
from jax import lax
from jax.experimental import pallas as pl
from jax.experimental.pallas import tpu as pltpu
from jax.experimental.pallas import tpu_sc as plsc

B = 16384
L = 20
D = 32
VOCAB = 1000000
Q = 128 // D
VQ = VOCAB // Q

_NC = 2
_NS = 16
_NW = _NC * _NS

_NEIGH_PW = (B * L) // _NW
_NODE_PW = B // _NW
_CHUNK = 256
_NCH_N = _NEIGH_PW // _CHUNK
_NCH_C = _NODE_PW // _CHUNK


def _sc_gather(iw128, uw128, ui_q, nodes_q):
    mesh = plsc.VectorSubcoreMesh(core_axis_name="c", subcore_axis_name="s")

    @functools.partial(
        pl.kernel,
        mesh=mesh,
        out_type=[
            jax.ShapeDtypeStruct((B * L, 128), jnp.float32),
            jax.ShapeDtypeStruct((B, 128), jnp.float32),
        ],
        scratch_types=[
            pltpu.VMEM((_CHUNK,), jnp.int32),
            pltpu.VMEM((_CHUNK,), jnp.int32),
            pltpu.VMEM((_CHUNK, 128), jnp.float32),
            pltpu.VMEM((_CHUNK, 128), jnp.float32),
            pltpu.SemaphoreType.DMA,
            pltpu.SemaphoreType.DMA,
        ],
    )
    def k(iw_hbm, uw_hbm, ui_hbm, nodes_hbm, neigh_out, node_out,
          idx0, idx1, rows0, rows1, sem0, sem1):
        wid = lax.axis_index("s") * _NC + lax.axis_index("c")
        idx_v = (idx0, idx1)
        rows_v = (rows0, rows1)
        sems = (sem0, sem1)

        def run(tbl, idx_hbm, out_hbm, base, nch):
            pltpu.sync_copy(idx_hbm.at[pl.ds(base, _CHUNK)], idx_v[0])
            pltpu.async_copy(tbl.at[idx_v[0]], rows_v[0], sems[0])
            for c in range(nch):
                nxt = (c + 1) % 2
                if c + 1 < nch:
                    pltpu.sync_copy(
                        idx_hbm.at[pl.ds(base + (c + 1) * _CHUNK, _CHUNK)],
                        idx_v[nxt])
                    pltpu.async_copy(tbl.at[idx_v[nxt]], rows_v[nxt],
                                     sems[nxt])
                cur = c % 2
                pltpu.make_async_copy(tbl.at[idx_v[cur]], rows_v[cur],
                                      sems[cur]).wait()
                pltpu.sync_copy(
                    rows_v[cur],
                    out_hbm.at[pl.ds(base + c * _CHUNK, _CHUNK)])

        run(iw_hbm, ui_hbm, neigh_out, wid * _NEIGH_PW, _NCH_N)
        run(uw_hbm, nodes_hbm, node_out, wid * _NODE_PW, _NCH_C)

    return k(iw128, uw128, ui_q, nodes_q)


_BT = 256


def _select32(block128, off):
    acc = jnp.where(off == 0, block128[:, 0 * D:1 * D], 0.0)
    acc += jnp.where(off == 1, block128[:, 1 * D:2 * D], 0.0)
    acc += jnp.where(off == 2, block128[:, 2 * D:3 * D], 0.0)
    acc += jnp.where(off == 3, block128[:, 3 * D:4 * D], 0.0)
    return acc


def _dense_body(neigh_ref, uio_ref, node_ref, no_ref, w1n_ref, w1c_ref,
                b1_ref, w2_ref, b2_ref, w3_ref, out_ref):
    neigh = _select32(neigh_ref[...], uio_ref[...] & (Q - 1))
    node = _select32(node_ref[...], no_ref[...] & (Q - 1))
    c1 = jnp.dot(node, w1c_ref[...],
                 preferred_element_type=jnp.float32) + b1_ref[...]
    h1 = jnp.dot(neigh, w1n_ref[...], preferred_element_type=jnp.float32)
    h1 = jnp.maximum(h1.reshape(_BT, L, D) + c1[:, None, :], 0.0)
    h2 = jnp.dot(h1.reshape(_BT * L, D), w2_ref[...],
                 preferred_element_type=jnp.float32) + b2_ref[...]
    h2 = jnp.maximum(h2, 0.0)
    logits = jnp.sum(h2.reshape(_BT, L, D) * w3_ref[...].reshape(1, 1, D),
                     axis=2)
    m = jnp.max(logits, axis=1, keepdims=True)
    e = jnp.exp(logits - m)
    att = e / jnp.sum(e, axis=1, keepdims=True)
    out_ref[...] = jnp.sum(neigh.reshape(_BT, L, D) * att[:, :, None], axis=1)


def _tc_dense(neigh128, ui_off, node128, n_off, w1n, w1c, b1, w2, b2, w3):
    grid = (B // _BT,)
    return pl.pallas_call(
        _dense_body,
        grid=grid,
        in_specs=[
            pl.BlockSpec((_BT * L, 128), lambda i: (i, 0)),
            pl.BlockSpec((_BT * L, 1), lambda i: (i, 0)),
            pl.BlockSpec((_BT, 128), lambda i: (i, 0)),
            pl.BlockSpec((_BT, 1), lambda i: (i, 0)),
            pl.BlockSpec((D, D), lambda i: (0, 0)),
            pl.BlockSpec((D, D), lambda i: (0, 0)),
            pl.BlockSpec((1, D), lambda i: (0, 0)),
            pl.BlockSpec((D, D), lambda i: (0, 0)),
            pl.BlockSpec((1, D), lambda i: (0, 0)),
            pl.BlockSpec((1, D), lambda i: (0, 0)),
        ],
        out_specs=pl.BlockSpec((_BT, D), lambda i: (i, 0)),
        out_shape=jax.ShapeDtypeStruct((B, D), jnp.float32),
        compiler_params=pltpu.CompilerParams(
            dimension_semantics=("arbitrary",)),
    )(neigh128, ui_off, node128, n_off, w1n, w1c, b1, w2, b2, w3)


def kernel(nodes, ui_network, ratings, u_weight, i_weight, W1, b1, W2, b2, W3, b3):
    ui_flat = ui_network.reshape(-1).astype(jnp.int32)
    nodes32 = nodes.astype(jnp.int32)
    iw128 = i_weight.reshape(VQ, 128)
    uw128 = u_weight.reshape(VQ, 128)
    neigh128, node128 = _sc_gather(iw128, uw128,
                                   ui_flat >> 2, nodes32 >> 2)
    w1n = W1[:, :D].T
    w1c = W1[:, D:].T
    w2 = W2.T
    return _tc_dense(neigh128, ui_flat.reshape(B * L, 1), node128,
                     nodes32.reshape(B, 1), w1n, w1c, b1.reshape(1, D),
                     w2, b2.reshape(1, D), W3.reshape(1, D))
